# Initial kernel scaffold; baseline (speedup 1.0000x reference)
import jax
import jax.numpy as jnp
from jax.experimental import pallas as pl


def kernel(x, edge_index, y, W1, b1, W2, b2, W3, b3, W4, b4):
    raise NotImplementedError("write your pallas kernel here")


# speedup vs baseline: 5.5337x; 5.5337x over previous
import functools

import jax
import jax.numpy as jnp
from jax import lax
from jax.experimental import pallas as pl
from jax.experimental.pallas import tpu as pltpu
from jax.experimental.pallas import tpu_sc as plsc

N = 10000
E = 160000
D_IN = 128
D_H = 1024
D_OUT = 40
D_OP = 128

MB = 256
GRID_M = (N + MB - 1) // MB

NSUB = 16
RPS = 624
WB_BLK = 48
TAIL0 = NSUB * RPS
TAILN = N - TAIL0

@functools.lru_cache(maxsize=1)
def _mesh():
    return plsc.VectorSubcoreMesh(core_axis_name="c", subcore_axis_name="s")


def _fill_const(ref, rows, cols, val):
    v = jnp.full((16,), val, jnp.float32)

    def body(i, c):
        for j in range(cols // 16):
            ref[i, pl.ds(j * 16, 16)] = v
        return c

    lax.fori_loop(0, rows, body, 0)


def _copy_rows(src_at, dst_at, tbuf, sid):
    for j in range(RPS // WB_BLK):
        r = pl.multiple_of(sid * RPS + j * WB_BLK, 8)
        pltpu.sync_copy(src_at(r, WB_BLK), tbuf)
        pltpu.sync_copy(tbuf, dst_at(r, WB_BLK))

    @pl.when(sid == NSUB - 1)
    def _():
        tb = tbuf.at[pl.ds(0, TAILN)]
        pltpu.sync_copy(src_at(TAIL0, TAILN), tb)
        pltpu.sync_copy(tb, dst_at(TAIL0, TAILN))


def _zero_rows(dst_at, zbuf, sid):
    for j in range(RPS // WB_BLK):
        r = pl.multiple_of(sid * RPS + j * WB_BLK, 8)
        pltpu.sync_copy(zbuf, dst_at(r, WB_BLK))

    @pl.when(sid == NSUB - 1)
    def _():
        pltpu.sync_copy(zbuf.at[pl.ds(0, TAILN)], dst_at(TAIL0, TAILN))


def _edge_pass(table, acc, sidx, didx, rows, sem, nblk):
    def body(i, c):
        pltpu.async_copy(table.at[sidx.at[i]], rows, sem).wait()
        pltpu.sync_copy(rows, acc.at[didx.at[i]], add=True)
        return c

    lax.fori_loop(0, nblk, body, 0)


def _sc_degree(dst3):
    NB, B = dst3.shape[1], dst3.shape[2]

    @functools.partial(
        pl.kernel,
        mesh=_mesh(),
        out_type=jax.ShapeDtypeStruct((2, N, D_OP), jnp.float32),
        scratch_types=[
            pltpu.VMEM_SHARED((N, D_OP), jnp.float32),
            pltpu.VMEM((NB, B), jnp.int32),
            pltpu.VMEM((B, D_OP), jnp.float32),
            pltpu.VMEM((WB_BLK, D_OP), jnp.float32),
        ],
    )
    def k(dst_h, out, acc, didx, ones, tbuf):
        core = lax.axis_index("c")
        sid = lax.axis_index("s")
        w = core * NSUB + sid
        pltpu.sync_copy(dst_h.at[w], didx)
        _fill_const(ones, B, D_OP, 1.0)
        _fill_const(tbuf, WB_BLK, D_OP, 0.0)
        _zero_rows(lambda r, n: acc.at[pl.ds(r, n)], tbuf, sid)
        plsc.subcore_barrier()

        def body(i, c):
            pltpu.sync_copy(ones, acc.at[didx.at[i]], add=True)
            return c

        lax.fori_loop(0, NB, body, 0)
        plsc.subcore_barrier()
        _copy_rows(lambda r, n: acc.at[pl.ds(r, n)],
                   lambda r, n: out.at[core, pl.ds(r, n)], tbuf, sid)

    return k(dst3)


def _sc_agg_chunks(tables, src3, dst3, C):
    K = len(tables)
    NB, B = src3.shape[1], src3.shape[2]

    @functools.partial(
        pl.kernel,
        mesh=_mesh(),
        out_type=jax.ShapeDtypeStruct((K, N, C), jnp.float32),
        scratch_types=[
            pltpu.VMEM_SHARED((N, C), jnp.float32),
            pltpu.VMEM((NB, B), jnp.int32),
            pltpu.VMEM((NB, B), jnp.int32),
            pltpu.VMEM((B, C), jnp.float32),
            pltpu.VMEM((WB_BLK, C), jnp.float32),
            pltpu.SemaphoreType.DMA,
        ],
    )
    def k(*refs):
        tabs = refs[:K]
        src_h, dst_h, out = refs[K], refs[K + 1], refs[K + 2]
        acc, sidx, didx, rows, tbuf, sem = refs[K + 3:]
        core = lax.axis_index("c")
        sid = lax.axis_index("s")
        pltpu.sync_copy(src_h.at[sid], sidx)
        pltpu.sync_copy(dst_h.at[sid], didx)
        for kk in range(K):
            @pl.when(core == (kk % 2))
            def _(kk=kk):
                _copy_rows(lambda r, n: tabs[kk].at[pl.ds(r, n)],
                           lambda r, n: acc.at[pl.ds(r, n)], tbuf, sid)
                plsc.subcore_barrier()
                _edge_pass(tabs[kk], acc, sidx, didx, rows, sem, NB)
                plsc.subcore_barrier()
                _copy_rows(lambda r, n: acc.at[pl.ds(r, n)],
                           lambda r, n: out.at[kk, pl.ds(r, n)], tbuf, sid)
                plsc.subcore_barrier()

    return k(*tables, src3, dst3)


def _sc_agg_split(table, src3, dst3):
    NB, B = src3.shape[1], src3.shape[2]

    @functools.partial(
        pl.kernel,
        mesh=_mesh(),
        out_type=jax.ShapeDtypeStruct((2, N, D_OP), jnp.float32),
        scratch_types=[
            pltpu.VMEM_SHARED((N, D_OP), jnp.float32),
            pltpu.VMEM((NB, B), jnp.int32),
            pltpu.VMEM((NB, B), jnp.int32),
            pltpu.VMEM((B, D_OP), jnp.float32),
            pltpu.VMEM((WB_BLK, D_OP), jnp.float32),
            pltpu.SemaphoreType.DMA,
        ],
    )
    def k(tab, src_h, dst_h, out, acc, sidx, didx, rows, tbuf, sem):
        core = lax.axis_index("c")
        sid = lax.axis_index("s")
        w = core * NSUB + sid
        pltpu.sync_copy(src_h.at[w], sidx)
        pltpu.sync_copy(dst_h.at[w], didx)
        _fill_const(tbuf, WB_BLK, D_OP, 0.0)
        _zero_rows(lambda r, n: acc.at[pl.ds(r, n)], tbuf, sid)
        plsc.subcore_barrier()
        _edge_pass(tab, acc, sidx, didx, rows, sem, NB)
        plsc.subcore_barrier()
        _copy_rows(lambda r, n: acc.at[pl.ds(r, n)],
                   lambda r, n: out.at[core, pl.ds(r, n)], tbuf, sid)

    return k(table, src3, dst3)


def _tc_prep(degp, x):
    def body(degp_ref, x_ref, dinv_ref, xn_ref):
        dp = degp_ref[...]
        deg = 1.0 + dp[0, :, 0:1] + dp[1, :, 0:1]
        dvc = lax.rsqrt(deg)
        dinv_ref[...] = jnp.broadcast_to(dvc, (MB, D_IN))
        xn_ref[...] = x_ref[...] * dvc

    return pl.pallas_call(
        body,
        grid=(GRID_M,),
        in_specs=[
            pl.BlockSpec((2, MB, D_OP), lambda m: (0, m, 0)),
            pl.BlockSpec((MB, D_IN), lambda m: (m, 0)),
        ],
        out_specs=[
            pl.BlockSpec((MB, D_IN), lambda m: (m, 0)),
            pl.BlockSpec((MB, D_IN), lambda m: (m, 0)),
        ],
        out_shape=[
            jax.ShapeDtypeStruct((N, D_IN), jnp.float32),
            jax.ShapeDtypeStruct((N, D_IN), jnp.float32),
        ],
    )(degp, x)


def _tc_layer1(p, xn, w, b, dinv):
    def body(p_ref, xn_ref, w_ref, b_ref, dinv_ref, *out_refs):
        pr = p_ref[...]
        dvc = dinv_ref[...][:, 0:1]
        a = (pr[0] + pr[1] + xn_ref[...]) * dvc
        h = jnp.dot(a, w_ref[...], preferred_element_type=jnp.float32)
        hn = jnp.maximum(h + b_ref[...], 0.0) * dvc
        for i in range(8):
            out_refs[i][...] = hn[:, i * 128:(i + 1) * 128]

    return pl.pallas_call(
        body,
        grid=(GRID_M,),
        in_specs=[
            pl.BlockSpec((2, MB, D_IN), lambda m: (0, m, 0)),
            pl.BlockSpec((MB, D_IN), lambda m: (m, 0)),
            pl.BlockSpec((D_IN, D_H), lambda m: (0, 0)),
            pl.BlockSpec((1, D_H), lambda m: (0, 0)),
            pl.BlockSpec((MB, D_IN), lambda m: (m, 0)),
        ],
        out_specs=[pl.BlockSpec((MB, 128), lambda m: (m, 0))] * 8,
        out_shape=[jax.ShapeDtypeStruct((N, 128), jnp.float32)] * 8,
    )(p, xn, w, b.reshape(1, D_H), dinv)


def _tc_layer(u, w, b, dinv):
    def body(u_ref, w_ref, b_ref, dinv_ref, *out_refs):
        ub = u_ref[...]
        a = jnp.concatenate([ub[i] for i in range(8)], axis=1)
        dvc = dinv_ref[...][:, 0:1]
        h = jnp.dot(a * dvc, w_ref[...], preferred_element_type=jnp.float32)
        hn = jnp.maximum(h + b_ref[...], 0.0) * dvc
        for i in range(8):
            out_refs[i][...] = hn[:, i * 128:(i + 1) * 128]

    return pl.pallas_call(
        body,
        grid=(GRID_M,),
        in_specs=[
            pl.BlockSpec((8, MB, 128), lambda m: (0, m, 0)),
            pl.BlockSpec((D_H, D_H), lambda m: (0, 0)),
            pl.BlockSpec((1, D_H), lambda m: (0, 0)),
            pl.BlockSpec((MB, D_IN), lambda m: (m, 0)),
        ],
        out_specs=[pl.BlockSpec((MB, 128), lambda m: (m, 0))] * 8,
        out_shape=[jax.ShapeDtypeStruct((N, 128), jnp.float32)] * 8,
    )(u, w, b.reshape(1, D_H), dinv)


def _tc_layer34(u, w3, b3, w4p, dinv):
    def body(u_ref, w3_ref, b3_ref, w4_ref, dinv_ref, out_ref):
        ub = u_ref[...]
        a = jnp.concatenate([ub[i] for i in range(8)], axis=1)
        dvc = dinv_ref[...][:, 0:1]
        h = jnp.dot(a * dvc, w3_ref[...], preferred_element_type=jnp.float32)
        h3 = jnp.maximum(h + b3_ref[...], 0.0)
        out_ref[...] = jnp.dot(
            h3, w4_ref[...], preferred_element_type=jnp.float32) * dvc

    return pl.pallas_call(
        body,
        grid=(GRID_M,),
        in_specs=[
            pl.BlockSpec((8, MB, 128), lambda m: (0, m, 0)),
            pl.BlockSpec((D_H, D_H), lambda m: (0, 0)),
            pl.BlockSpec((1, D_H), lambda m: (0, 0)),
            pl.BlockSpec((D_H, D_OP), lambda m: (0, 0)),
            pl.BlockSpec((MB, D_IN), lambda m: (m, 0)),
        ],
        out_specs=pl.BlockSpec((MB, D_OP), lambda m: (m, 0)),
        out_shape=jax.ShapeDtypeStruct((N, D_OP), jnp.float32),
    )(u, w3, b3.reshape(1, D_H), w4p, dinv)


def _tc_final(p, hn4, dinv, b4p, y2):
    def body(p_ref, hn4_ref, dinv_ref, b4_ref, y_ref, lg_ref, ls_ref):
        pr = p_ref[...]
        dvc = dinv_ref[...][:, 0:1]
        l = (pr[0] + pr[1] + hn4_ref[...]) * dvc + b4_ref[...]
        lg_ref[...] = l
        col = lax.broadcasted_iota(jnp.int32, (MB, D_OP), 1)
        lm = jnp.where(col < D_OUT, l, -1e30)
        mx = jnp.max(lm, axis=1, keepdims=True)
        lse = jnp.log(jnp.sum(jnp.exp(lm - mx), axis=1, keepdims=True)) + mx
        ysel = jnp.sum(jnp.where(col == y_ref[...], lm, 0.0), axis=1,
                       keepdims=True)
        nll = lse - ysel
        rid = lax.broadcasted_iota(jnp.int32, (MB, 1), 0) + pl.program_id(0) * MB
        contrib = jnp.sum(jnp.where(rid < N, nll, 0.0), axis=(0, 1),
                          keepdims=True)

        @pl.when(pl.program_id(0) == 0)
        def _():
            ls_ref[...] = jnp.zeros((1, 1), jnp.float32)

        ls_ref[...] += contrib

    return pl.pallas_call(
        body,
        grid=(GRID_M,),
        in_specs=[
            pl.BlockSpec((2, MB, D_OP), lambda m: (0, m, 0)),
            pl.BlockSpec((MB, D_OP), lambda m: (m, 0)),
            pl.BlockSpec((MB, D_IN), lambda m: (m, 0)),
            pl.BlockSpec((1, D_OP), lambda m: (0, 0)),
            pl.BlockSpec((MB, 1), lambda m: (m, 0)),
        ],
        out_specs=[
            pl.BlockSpec((MB, D_OP), lambda m: (m, 0)),
            pl.BlockSpec((1, 1), lambda m: (0, 0)),
        ],
        out_shape=[
            jax.ShapeDtypeStruct((N, D_OP), jnp.float32),
            jax.ShapeDtypeStruct((1, 1), jnp.float32),
        ],
    )(p, hn4, dinv, b4p, y2)


def kernel(x, edge_index, y, W1, b1, W2, b2, W3, b3, W4, b4):
    src = edge_index[0]
    dst = edge_index[1]
    src80 = src.reshape(NSUB, E // NSUB // 80, 80)
    dst80 = dst.reshape(NSUB, E // NSUB // 80, 80)
    src40 = src.reshape(2 * NSUB, E // (2 * NSUB) // 40, 40)
    dst40 = dst.reshape(2 * NSUB, E // (2 * NSUB) // 40, 40)
    w4p = jnp.pad(W4, ((0, 0), (0, D_OP - D_OUT)))
    b4p = jnp.pad(b4, (0, D_OP - D_OUT)).reshape(1, D_OP)
    y2 = y.reshape(N, 1)

    degp = _sc_degree(dst40)
    dinv, xn = _tc_prep(degp, x)
    p0 = _sc_agg_split(xn, src40, dst40)
    hn1 = _tc_layer1(p0, xn, W1, b1, dinv)
    u1 = _sc_agg_chunks(hn1, src80, dst80, 128)
    hn2 = _tc_layer(u1, W2, b2, dinv)
    u2 = _sc_agg_chunks(hn2, src80, dst80, 128)
    hn4 = _tc_layer34(u2, W3, b3, w4p, dinv)
    p = _sc_agg_split(hn4, src40, dst40)
    lg, lsum = _tc_final(p, hn4, dinv, b4p, y2)
    logits = lg[:, :D_OUT]
    loss = lsum[0, 0] / jnp.float32(N)
    return logits, loss


# speedup vs baseline: 6.6840x; 1.2079x over previous
import functools

import jax
import jax.numpy as jnp
from jax import lax
from jax.experimental import pallas as pl
from jax.experimental.pallas import tpu as pltpu
from jax.experimental.pallas import tpu_sc as plsc

N = 10000
E = 160000
D_IN = 128
D_H = 1024
D_OUT = 40
D_OP = 128

MB = 256
GRID_M = (N + MB - 1) // MB

NSUB = 16
RPS = 624
WB_BLK = 48
TAIL0 = NSUB * RPS
TAILN = N - TAIL0

@functools.lru_cache(maxsize=1)
def _mesh():
    return plsc.VectorSubcoreMesh(core_axis_name="c", subcore_axis_name="s")


def _fill_const(ref, rows, cols, val):
    v = jnp.full((16,), val, jnp.float32)

    def body(i, c):
        for j in range(cols // 16):
            ref[i, pl.ds(j * 16, 16)] = v
        return c

    lax.fori_loop(0, rows, body, 0)


def _copy_rows(src_at, dst_at, tbuf, sid):
    for j in range(RPS // WB_BLK):
        r = pl.multiple_of(sid * RPS + j * WB_BLK, 8)
        pltpu.sync_copy(src_at(r, WB_BLK), tbuf)
        pltpu.sync_copy(tbuf, dst_at(r, WB_BLK))

    @pl.when(sid == NSUB - 1)
    def _():
        tb = tbuf.at[pl.ds(0, TAILN)]
        pltpu.sync_copy(src_at(TAIL0, TAILN), tb)
        pltpu.sync_copy(tb, dst_at(TAIL0, TAILN))


def _zero_rows(dst_at, zbuf, sid):
    for j in range(RPS // WB_BLK):
        r = pl.multiple_of(sid * RPS + j * WB_BLK, 8)
        pltpu.sync_copy(zbuf, dst_at(r, WB_BLK))

    @pl.when(sid == NSUB - 1)
    def _():
        pltpu.sync_copy(zbuf.at[pl.ds(0, TAILN)], dst_at(TAIL0, TAILN))


def _edge_pass(table, acc, sidx, didx, rows0, rows1, sem0, sem1, nblk):
    assert nblk % 2 == 1

    def start(j, buf, sem):
        pltpu.async_copy(table.at[sidx.at[j]], buf, sem)

    def wait(j, buf, sem):
        pltpu.make_async_copy(table.at[sidx.at[j]], buf, sem).wait()

    start(0, rows0, sem0)

    @pl.loop(0, nblk - 1, step=2)
    def _(j):
        wait(j, rows0, sem0)
        start(j + 1, rows1, sem1)
        pltpu.sync_copy(rows0, acc.at[didx.at[j]], add=True)
        wait(j + 1, rows1, sem1)
        start(j + 2, rows0, sem0)
        pltpu.sync_copy(rows1, acc.at[didx.at[j + 1]], add=True)

    wait(nblk - 1, rows0, sem0)
    pltpu.sync_copy(rows0, acc.at[didx.at[nblk - 1]], add=True)


def _sc_degree(dst3):
    NB, B = dst3.shape[1], dst3.shape[2]

    @functools.partial(
        pl.kernel,
        mesh=_mesh(),
        out_type=jax.ShapeDtypeStruct((2, N, D_OP), jnp.float32),
        scratch_types=[
            pltpu.VMEM_SHARED((N, D_OP), jnp.float32),
            pltpu.VMEM((NB, B), jnp.int32),
            pltpu.VMEM((B, D_OP), jnp.float32),
            pltpu.VMEM((WB_BLK, D_OP), jnp.float32),
        ],
    )
    def k(dst_h, out, acc, didx, ones, tbuf):
        core = lax.axis_index("c")
        sid = lax.axis_index("s")
        w = core * NSUB + sid
        pltpu.sync_copy(dst_h.at[w], didx)
        _fill_const(ones, B, D_OP, 1.0)
        _fill_const(tbuf, WB_BLK, D_OP, 0.0)
        _zero_rows(lambda r, n: acc.at[pl.ds(r, n)], tbuf, sid)
        plsc.subcore_barrier()

        def body(i, c):
            pltpu.sync_copy(ones, acc.at[didx.at[i]], add=True)
            return c

        lax.fori_loop(0, NB, body, 0)
        plsc.subcore_barrier()
        _copy_rows(lambda r, n: acc.at[pl.ds(r, n)],
                   lambda r, n: out.at[core, pl.ds(r, n)], tbuf, sid)

    return k(dst3)


def _sc_agg_chunks(tables, src3, dst3, C):
    K = len(tables)
    NP, NB, B = src3.shape[1], src3.shape[2], src3.shape[3]

    @functools.partial(
        pl.kernel,
        mesh=_mesh(),
        out_type=jax.ShapeDtypeStruct((K, N, C), jnp.float32),
        scratch_types=[
            pltpu.VMEM_SHARED((N, C), jnp.float32),
            pltpu.VMEM((NB, B), jnp.int32),
            pltpu.VMEM((NB, B), jnp.int32),
            pltpu.VMEM((B, C), jnp.float32),
            pltpu.VMEM((B, C), jnp.float32),
            pltpu.SemaphoreType.DMA,
            pltpu.SemaphoreType.DMA,
        ],
    )
    def k(*refs):
        tabs = refs[:K]
        src_h, dst_h, out = refs[K], refs[K + 1], refs[K + 2]
        acc, sidx, didx, rows0, rows1, sem0, sem1 = refs[K + 3:]
        core = lax.axis_index("c")
        sid = lax.axis_index("s")
        tbuf = rows1.at[pl.ds(0, WB_BLK)]
        for kk in range(K):
            @pl.when(core == (kk % 2))
            def _(kk=kk):
                _copy_rows(lambda r, n: tabs[kk].at[pl.ds(r, n)],
                           lambda r, n: acc.at[pl.ds(r, n)], tbuf, sid)
                plsc.subcore_barrier()
                for p in range(NP):
                    pltpu.sync_copy(src_h.at[sid, p], sidx)
                    pltpu.sync_copy(dst_h.at[sid, p], didx)
                    _edge_pass(tabs[kk], acc, sidx, didx, rows0, rows1,
                               sem0, sem1, NB)
                plsc.subcore_barrier()
                _copy_rows(lambda r, n: acc.at[pl.ds(r, n)],
                           lambda r, n: out.at[kk, pl.ds(r, n)], tbuf, sid)
                plsc.subcore_barrier()

    return k(*tables, src3, dst3)


def _sc_agg_split(table, src3, dst3):
    NB, B = src3.shape[1], src3.shape[2]

    @functools.partial(
        pl.kernel,
        mesh=_mesh(),
        out_type=jax.ShapeDtypeStruct((2, N, D_OP), jnp.float32),
        scratch_types=[
            pltpu.VMEM_SHARED((N, D_OP), jnp.float32),
            pltpu.VMEM((NB, B), jnp.int32),
            pltpu.VMEM((NB, B), jnp.int32),
            pltpu.VMEM((B, D_OP), jnp.float32),
            pltpu.VMEM((B, D_OP), jnp.float32),
            pltpu.VMEM((WB_BLK, D_OP), jnp.float32),
            pltpu.SemaphoreType.DMA,
            pltpu.SemaphoreType.DMA,
        ],
    )
    def k(tab, src_h, dst_h, out, acc, sidx, didx, rows0, rows1, tbuf,
          sem0, sem1):
        core = lax.axis_index("c")
        sid = lax.axis_index("s")
        w = core * NSUB + sid
        pltpu.sync_copy(src_h.at[w], sidx)
        pltpu.sync_copy(dst_h.at[w], didx)
        _fill_const(tbuf, WB_BLK, D_OP, 0.0)
        _zero_rows(lambda r, n: acc.at[pl.ds(r, n)], tbuf, sid)
        plsc.subcore_barrier()
        _edge_pass(tab, acc, sidx, didx, rows0, rows1, sem0, sem1, NB)
        plsc.subcore_barrier()
        _copy_rows(lambda r, n: acc.at[pl.ds(r, n)],
                   lambda r, n: out.at[core, pl.ds(r, n)], tbuf, sid)

    return k(table, src3, dst3)


def _tc_prep(degp, x):
    def body(degp_ref, x_ref, dinv_ref, xn_ref):
        dp = degp_ref[...]
        deg = 1.0 + dp[0, :, 0:1] + dp[1, :, 0:1]
        dvc = lax.rsqrt(deg)
        dinv_ref[...] = jnp.broadcast_to(dvc, (MB, D_IN))
        xn_ref[...] = x_ref[...] * dvc

    return pl.pallas_call(
        body,
        grid=(GRID_M,),
        in_specs=[
            pl.BlockSpec((2, MB, D_OP), lambda m: (0, m, 0)),
            pl.BlockSpec((MB, D_IN), lambda m: (m, 0)),
        ],
        out_specs=[
            pl.BlockSpec((MB, D_IN), lambda m: (m, 0)),
            pl.BlockSpec((MB, D_IN), lambda m: (m, 0)),
        ],
        out_shape=[
            jax.ShapeDtypeStruct((N, D_IN), jnp.float32),
            jax.ShapeDtypeStruct((N, D_IN), jnp.float32),
        ],
    )(degp, x)


def _tc_layer1(p, xn, w, b, dinv):
    def body(p_ref, xn_ref, w_ref, b_ref, dinv_ref, *out_refs):
        pr = p_ref[...]
        dvc = dinv_ref[...][:, 0:1]
        a = (pr[0] + pr[1] + xn_ref[...]) * dvc
        h = jnp.dot(a, w_ref[...], preferred_element_type=jnp.float32)
        hn = jnp.maximum(h + b_ref[...], 0.0) * dvc
        for i in range(8):
            out_refs[i][...] = hn[:, i * 128:(i + 1) * 128]

    return pl.pallas_call(
        body,
        grid=(GRID_M,),
        in_specs=[
            pl.BlockSpec((2, MB, D_IN), lambda m: (0, m, 0)),
            pl.BlockSpec((MB, D_IN), lambda m: (m, 0)),
            pl.BlockSpec((D_IN, D_H), lambda m: (0, 0)),
            pl.BlockSpec((1, D_H), lambda m: (0, 0)),
            pl.BlockSpec((MB, D_IN), lambda m: (m, 0)),
        ],
        out_specs=[pl.BlockSpec((MB, 128), lambda m: (m, 0))] * 8,
        out_shape=[jax.ShapeDtypeStruct((N, 128), jnp.float32)] * 8,
    )(p, xn, w, b.reshape(1, D_H), dinv)


def _tc_layer(u, w, b, dinv):
    def body(u_ref, w_ref, b_ref, dinv_ref, *out_refs):
        ub = u_ref[...]
        a = jnp.concatenate([ub[i] for i in range(8)], axis=1)
        dvc = dinv_ref[...][:, 0:1]
        h = jnp.dot(a * dvc, w_ref[...], preferred_element_type=jnp.float32)
        hn = jnp.maximum(h + b_ref[...], 0.0) * dvc
        for i in range(8):
            out_refs[i][...] = hn[:, i * 128:(i + 1) * 128]

    return pl.pallas_call(
        body,
        grid=(GRID_M,),
        in_specs=[
            pl.BlockSpec((8, MB, 128), lambda m: (0, m, 0)),
            pl.BlockSpec((D_H, D_H), lambda m: (0, 0)),
            pl.BlockSpec((1, D_H), lambda m: (0, 0)),
            pl.BlockSpec((MB, D_IN), lambda m: (m, 0)),
        ],
        out_specs=[pl.BlockSpec((MB, 128), lambda m: (m, 0))] * 8,
        out_shape=[jax.ShapeDtypeStruct((N, 128), jnp.float32)] * 8,
    )(u, w, b.reshape(1, D_H), dinv)


def _tc_layer34(u, w3, b3, w4p, dinv):
    def body(u_ref, w3_ref, b3_ref, w4_ref, dinv_ref, out_ref):
        ub = u_ref[...]
        a = jnp.concatenate([ub[i] for i in range(8)], axis=1)
        dvc = dinv_ref[...][:, 0:1]
        h = jnp.dot(a * dvc, w3_ref[...], preferred_element_type=jnp.float32)
        h3 = jnp.maximum(h + b3_ref[...], 0.0)
        out_ref[...] = jnp.dot(
            h3, w4_ref[...], preferred_element_type=jnp.float32) * dvc

    return pl.pallas_call(
        body,
        grid=(GRID_M,),
        in_specs=[
            pl.BlockSpec((8, MB, 128), lambda m: (0, m, 0)),
            pl.BlockSpec((D_H, D_H), lambda m: (0, 0)),
            pl.BlockSpec((1, D_H), lambda m: (0, 0)),
            pl.BlockSpec((D_H, D_OP), lambda m: (0, 0)),
            pl.BlockSpec((MB, D_IN), lambda m: (m, 0)),
        ],
        out_specs=pl.BlockSpec((MB, D_OP), lambda m: (m, 0)),
        out_shape=jax.ShapeDtypeStruct((N, D_OP), jnp.float32),
    )(u, w3, b3.reshape(1, D_H), w4p, dinv)


def _tc_final(p, hn4, dinv, b4p, y2):
    def body(p_ref, hn4_ref, dinv_ref, b4_ref, y_ref, lg_ref, ls_ref):
        pr = p_ref[...]
        dvc = dinv_ref[...][:, 0:1]
        l = (pr[0] + pr[1] + hn4_ref[...]) * dvc + b4_ref[...]
        lg_ref[...] = l
        col = lax.broadcasted_iota(jnp.int32, (MB, D_OP), 1)
        lm = jnp.where(col < D_OUT, l, -1e30)
        mx = jnp.max(lm, axis=1, keepdims=True)
        lse = jnp.log(jnp.sum(jnp.exp(lm - mx), axis=1, keepdims=True)) + mx
        ysel = jnp.sum(jnp.where(col == y_ref[...], lm, 0.0), axis=1,
                       keepdims=True)
        nll = lse - ysel
        rid = lax.broadcasted_iota(jnp.int32, (MB, 1), 0) + pl.program_id(0) * MB
        contrib = jnp.sum(jnp.where(rid < N, nll, 0.0), axis=(0, 1),
                          keepdims=True)

        @pl.when(pl.program_id(0) == 0)
        def _():
            ls_ref[...] = jnp.zeros((1, 1), jnp.float32)

        ls_ref[...] += contrib

    return pl.pallas_call(
        body,
        grid=(GRID_M,),
        in_specs=[
            pl.BlockSpec((2, MB, D_OP), lambda m: (0, m, 0)),
            pl.BlockSpec((MB, D_OP), lambda m: (m, 0)),
            pl.BlockSpec((MB, D_IN), lambda m: (m, 0)),
            pl.BlockSpec((1, D_OP), lambda m: (0, 0)),
            pl.BlockSpec((MB, 1), lambda m: (m, 0)),
        ],
        out_specs=[
            pl.BlockSpec((MB, D_OP), lambda m: (m, 0)),
            pl.BlockSpec((1, 1), lambda m: (0, 0)),
        ],
        out_shape=[
            jax.ShapeDtypeStruct((N, D_OP), jnp.float32),
            jax.ShapeDtypeStruct((1, 1), jnp.float32),
        ],
    )(p, hn4, dinv, b4p, y2)


def kernel(x, edge_index, y, W1, b1, W2, b2, W3, b3, W4, b4):
    src = edge_index[0]
    dst = edge_index[1]
    src80 = src.reshape(NSUB, 5, E // NSUB // 80 // 5, 80)
    dst80 = dst.reshape(NSUB, 5, E // NSUB // 80 // 5, 80)
    src40 = src.reshape(2 * NSUB, E // (2 * NSUB) // 40, 40)
    dst40 = dst.reshape(2 * NSUB, E // (2 * NSUB) // 40, 40)
    w4p = jnp.pad(W4, ((0, 0), (0, D_OP - D_OUT)))
    b4p = jnp.pad(b4, (0, D_OP - D_OUT)).reshape(1, D_OP)
    y2 = y.reshape(N, 1)

    degp = _sc_degree(dst40)
    dinv, xn = _tc_prep(degp, x)
    p0 = _sc_agg_split(xn, src40, dst40)
    hn1 = _tc_layer1(p0, xn, W1, b1, dinv)
    u1 = _sc_agg_chunks(hn1, src80, dst80, 128)
    hn2 = _tc_layer(u1, W2, b2, dinv)
    u2 = _sc_agg_chunks(hn2, src80, dst80, 128)
    hn4 = _tc_layer34(u2, W3, b3, w4p, dinv)
    p = _sc_agg_split(hn4, src40, dst40)
    lg, lsum = _tc_final(p, hn4, dinv, b4p, y2)
    logits = lg[:, :D_OUT]
    loss = lsum[0, 0] / jnp.float32(N)
    return logits, loss


# speedup vs baseline: 6.8619x; 1.0266x over previous
import functools

import jax
import jax.numpy as jnp
from jax import lax
from jax.experimental import pallas as pl
from jax.experimental.pallas import tpu as pltpu
from jax.experimental.pallas import tpu_sc as plsc

N = 10000
E = 160000
D_IN = 128
D_H = 1024
D_OUT = 40
D_OP = 128

MB = 256
GRID_M = (N + MB - 1) // MB

NSUB = 16
RPS = 624
WB_BLK = 48
TAIL0 = NSUB * RPS
TAILN = N - TAIL0

@functools.lru_cache(maxsize=1)
def _mesh():
    return plsc.VectorSubcoreMesh(core_axis_name="c", subcore_axis_name="s")


def _fill_const(ref, rows, cols, val):
    v = jnp.full((16,), val, jnp.float32)

    def body(i, c):
        for j in range(cols // 16):
            ref[i, pl.ds(j * 16, 16)] = v
        return c

    lax.fori_loop(0, rows, body, 0)


def _copy_rows(src_at, dst_at, tbuf, sid):
    for j in range(RPS // WB_BLK):
        r = pl.multiple_of(sid * RPS + j * WB_BLK, 8)
        pltpu.sync_copy(src_at(r, WB_BLK), tbuf)
        pltpu.sync_copy(tbuf, dst_at(r, WB_BLK))

    @pl.when(sid == NSUB - 1)
    def _():
        tb = tbuf.at[pl.ds(0, TAILN)]
        pltpu.sync_copy(src_at(TAIL0, TAILN), tb)
        pltpu.sync_copy(tb, dst_at(TAIL0, TAILN))


def _zero_rows(dst_at, zbuf, sid):
    for j in range(RPS // WB_BLK):
        r = pl.multiple_of(sid * RPS + j * WB_BLK, 8)
        pltpu.sync_copy(zbuf, dst_at(r, WB_BLK))

    @pl.when(sid == NSUB - 1)
    def _():
        pltpu.sync_copy(zbuf.at[pl.ds(0, TAILN)], dst_at(TAIL0, TAILN))


def _edge_pass(table, acc, sidx, didx, rows0, rows1, g0, g1, s0, s1, nblk):
    assert nblk % 2 == 1

    def sg(j, buf, sem):
        pltpu.async_copy(table.at[sidx.at[j]], buf, sem)

    def wg(j, buf, sem):
        pltpu.make_async_copy(table.at[sidx.at[j]], buf, sem).wait()

    def ssc(j, buf, sem):
        pltpu.async_copy(buf, acc.at[didx.at[j]], sem, add=True)

    def wsc(j, buf, sem):
        pltpu.make_async_copy(buf, acc.at[didx.at[j]], sem).wait()

    sg(0, rows0, g0)
    sg(1, rows1, g1)

    @pl.loop(0, nblk - 1, step=2)
    def _(j):
        wg(j, rows0, g0)
        ssc(j, rows0, s0)
        wg(j + 1, rows1, g1)
        ssc(j + 1, rows1, s1)
        wsc(j, rows0, s0)
        sg(j + 2, rows0, g0)
        wsc(j + 1, rows1, s1)

        @pl.when(j + 3 < nblk)
        def _():
            sg(j + 3, rows1, g1)

    wg(nblk - 1, rows0, g0)
    ssc(nblk - 1, rows0, s0)
    wsc(nblk - 1, rows0, s0)


def _sc_degree(dst3):
    NB, B = dst3.shape[1], dst3.shape[2]

    @functools.partial(
        pl.kernel,
        mesh=_mesh(),
        out_type=jax.ShapeDtypeStruct((2, N, D_OP), jnp.float32),
        scratch_types=[
            pltpu.VMEM_SHARED((N, D_OP), jnp.float32),
            pltpu.VMEM((NB, B), jnp.int32),
            pltpu.VMEM((B, D_OP), jnp.float32),
            pltpu.VMEM((WB_BLK, D_OP), jnp.float32),
        ],
    )
    def k(dst_h, out, acc, didx, ones, tbuf):
        core = lax.axis_index("c")
        sid = lax.axis_index("s")
        w = core * NSUB + sid
        pltpu.sync_copy(dst_h.at[w], didx)
        _fill_const(ones, B, D_OP, 1.0)
        _fill_const(tbuf, WB_BLK, D_OP, 0.0)
        _zero_rows(lambda r, n: acc.at[pl.ds(r, n)], tbuf, sid)
        plsc.subcore_barrier()

        def body(i, c):
            pltpu.sync_copy(ones, acc.at[didx.at[i]], add=True)
            return c

        lax.fori_loop(0, NB, body, 0)
        plsc.subcore_barrier()
        _copy_rows(lambda r, n: acc.at[pl.ds(r, n)],
                   lambda r, n: out.at[core, pl.ds(r, n)], tbuf, sid)

    return k(dst3)


def _sc_agg_chunks(tables, src3, dst3, C):
    K = len(tables)
    NP, NB, B = src3.shape[1], src3.shape[2], src3.shape[3]

    @functools.partial(
        pl.kernel,
        mesh=_mesh(),
        out_type=jax.ShapeDtypeStruct((K, N, C), jnp.float32),
        scratch_types=[
            pltpu.VMEM_SHARED((N, C), jnp.float32),
            pltpu.VMEM((NB, B), jnp.int32),
            pltpu.VMEM((NB, B), jnp.int32),
            pltpu.VMEM((B, C), jnp.float32),
            pltpu.VMEM((B, C), jnp.float32),
            pltpu.SemaphoreType.DMA,
            pltpu.SemaphoreType.DMA,
            pltpu.SemaphoreType.DMA,
            pltpu.SemaphoreType.DMA,
        ],
    )
    def k(*refs):
        tabs = refs[:K]
        src_h, dst_h, out = refs[K], refs[K + 1], refs[K + 2]
        acc, sidx, didx, rows0, rows1, g0, g1, s0, s1 = refs[K + 3:]
        core = lax.axis_index("c")
        sid = lax.axis_index("s")
        tbuf = rows1.at[pl.ds(0, WB_BLK)]
        for kk in range(K):
            @pl.when(core == (kk % 2))
            def _(kk=kk):
                _copy_rows(lambda r, n: tabs[kk].at[pl.ds(r, n)],
                           lambda r, n: acc.at[pl.ds(r, n)], tbuf, sid)
                plsc.subcore_barrier()
                for p in range(NP):
                    pltpu.sync_copy(src_h.at[sid, p], sidx)
                    pltpu.sync_copy(dst_h.at[sid, p], didx)
                    _edge_pass(tabs[kk], acc, sidx, didx, rows0, rows1,
                               g0, g1, s0, s1, NB)
                plsc.subcore_barrier()
                _copy_rows(lambda r, n: acc.at[pl.ds(r, n)],
                           lambda r, n: out.at[kk, pl.ds(r, n)], tbuf, sid)
                plsc.subcore_barrier()

    return k(*tables, src3, dst3)


def _sc_agg_split(table, src3, dst3):
    NB, B = src3.shape[1], src3.shape[2]

    @functools.partial(
        pl.kernel,
        mesh=_mesh(),
        out_type=jax.ShapeDtypeStruct((2, N, D_OP), jnp.float32),
        scratch_types=[
            pltpu.VMEM_SHARED((N, D_OP), jnp.float32),
            pltpu.VMEM((NB, B), jnp.int32),
            pltpu.VMEM((NB, B), jnp.int32),
            pltpu.VMEM((B, D_OP), jnp.float32),
            pltpu.VMEM((B, D_OP), jnp.float32),
            pltpu.VMEM((WB_BLK, D_OP), jnp.float32),
            pltpu.SemaphoreType.DMA,
            pltpu.SemaphoreType.DMA,
            pltpu.SemaphoreType.DMA,
            pltpu.SemaphoreType.DMA,
        ],
    )
    def k(tab, src_h, dst_h, out, acc, sidx, didx, rows0, rows1, tbuf,
          g0, g1, s0, s1):
        core = lax.axis_index("c")
        sid = lax.axis_index("s")
        w = core * NSUB + sid
        pltpu.sync_copy(src_h.at[w], sidx)
        pltpu.sync_copy(dst_h.at[w], didx)
        _fill_const(tbuf, WB_BLK, D_OP, 0.0)
        _zero_rows(lambda r, n: acc.at[pl.ds(r, n)], tbuf, sid)
        plsc.subcore_barrier()
        _edge_pass(tab, acc, sidx, didx, rows0, rows1, g0, g1, s0, s1, NB)
        plsc.subcore_barrier()
        _copy_rows(lambda r, n: acc.at[pl.ds(r, n)],
                   lambda r, n: out.at[core, pl.ds(r, n)], tbuf, sid)

    return k(table, src3, dst3)


def _tc_prep(degp, x):
    def body(degp_ref, x_ref, dinv_ref, xn_ref):
        dp = degp_ref[...]
        deg = 1.0 + dp[0, :, 0:1] + dp[1, :, 0:1]
        dvc = lax.rsqrt(deg)
        dinv_ref[...] = jnp.broadcast_to(dvc, (MB, D_IN))
        xn_ref[...] = x_ref[...] * dvc

    return pl.pallas_call(
        body,
        grid=(GRID_M,),
        in_specs=[
            pl.BlockSpec((2, MB, D_OP), lambda m: (0, m, 0)),
            pl.BlockSpec((MB, D_IN), lambda m: (m, 0)),
        ],
        out_specs=[
            pl.BlockSpec((MB, D_IN), lambda m: (m, 0)),
            pl.BlockSpec((MB, D_IN), lambda m: (m, 0)),
        ],
        out_shape=[
            jax.ShapeDtypeStruct((N, D_IN), jnp.float32),
            jax.ShapeDtypeStruct((N, D_IN), jnp.float32),
        ],
    )(degp, x)


def _tc_layer1(p, xn, w, b, dinv):
    def body(p_ref, xn_ref, w_ref, b_ref, dinv_ref, *out_refs):
        pr = p_ref[...]
        dvc = dinv_ref[...][:, 0:1]
        a = (pr[0] + pr[1] + xn_ref[...]) * dvc
        h = jnp.dot(a, w_ref[...], preferred_element_type=jnp.float32)
        hn = jnp.maximum(h + b_ref[...], 0.0) * dvc
        for i in range(8):
            out_refs[i][...] = hn[:, i * 128:(i + 1) * 128]

    return pl.pallas_call(
        body,
        grid=(GRID_M,),
        in_specs=[
            pl.BlockSpec((2, MB, D_IN), lambda m: (0, m, 0)),
            pl.BlockSpec((MB, D_IN), lambda m: (m, 0)),
            pl.BlockSpec((D_IN, D_H), lambda m: (0, 0)),
            pl.BlockSpec((1, D_H), lambda m: (0, 0)),
            pl.BlockSpec((MB, D_IN), lambda m: (m, 0)),
        ],
        out_specs=[pl.BlockSpec((MB, 128), lambda m: (m, 0))] * 8,
        out_shape=[jax.ShapeDtypeStruct((N, 128), jnp.float32)] * 8,
    )(p, xn, w, b.reshape(1, D_H), dinv)


def _tc_layer(u, w, b, dinv):
    def body(u_ref, w_ref, b_ref, dinv_ref, *out_refs):
        ub = u_ref[...]
        a = jnp.concatenate([ub[i] for i in range(8)], axis=1)
        dvc = dinv_ref[...][:, 0:1]
        h = jnp.dot(a * dvc, w_ref[...], preferred_element_type=jnp.float32)
        hn = jnp.maximum(h + b_ref[...], 0.0) * dvc
        for i in range(8):
            out_refs[i][...] = hn[:, i * 128:(i + 1) * 128]

    return pl.pallas_call(
        body,
        grid=(GRID_M,),
        in_specs=[
            pl.BlockSpec((8, MB, 128), lambda m: (0, m, 0)),
            pl.BlockSpec((D_H, D_H), lambda m: (0, 0)),
            pl.BlockSpec((1, D_H), lambda m: (0, 0)),
            pl.BlockSpec((MB, D_IN), lambda m: (m, 0)),
        ],
        out_specs=[pl.BlockSpec((MB, 128), lambda m: (m, 0))] * 8,
        out_shape=[jax.ShapeDtypeStruct((N, 128), jnp.float32)] * 8,
    )(u, w, b.reshape(1, D_H), dinv)


def _tc_layer34(u, w3, b3, w4p, dinv):
    def body(u_ref, w3_ref, b3_ref, w4_ref, dinv_ref, out_ref):
        ub = u_ref[...]
        a = jnp.concatenate([ub[i] for i in range(8)], axis=1)
        dvc = dinv_ref[...][:, 0:1]
        h = jnp.dot(a * dvc, w3_ref[...], preferred_element_type=jnp.float32)
        h3 = jnp.maximum(h + b3_ref[...], 0.0)
        out_ref[...] = jnp.dot(
            h3, w4_ref[...], preferred_element_type=jnp.float32) * dvc

    return pl.pallas_call(
        body,
        grid=(GRID_M,),
        in_specs=[
            pl.BlockSpec((8, MB, 128), lambda m: (0, m, 0)),
            pl.BlockSpec((D_H, D_H), lambda m: (0, 0)),
            pl.BlockSpec((1, D_H), lambda m: (0, 0)),
            pl.BlockSpec((D_H, D_OP), lambda m: (0, 0)),
            pl.BlockSpec((MB, D_IN), lambda m: (m, 0)),
        ],
        out_specs=pl.BlockSpec((MB, D_OP), lambda m: (m, 0)),
        out_shape=jax.ShapeDtypeStruct((N, D_OP), jnp.float32),
    )(u, w3, b3.reshape(1, D_H), w4p, dinv)


def _tc_final(p, hn4, dinv, b4p, y2):
    def body(p_ref, hn4_ref, dinv_ref, b4_ref, y_ref, lg_ref, ls_ref):
        pr = p_ref[...]
        dvc = dinv_ref[...][:, 0:1]
        l = (pr[0] + pr[1] + hn4_ref[...]) * dvc + b4_ref[...]
        lg_ref[...] = l
        col = lax.broadcasted_iota(jnp.int32, (MB, D_OP), 1)
        lm = jnp.where(col < D_OUT, l, -1e30)
        mx = jnp.max(lm, axis=1, keepdims=True)
        lse = jnp.log(jnp.sum(jnp.exp(lm - mx), axis=1, keepdims=True)) + mx
        ysel = jnp.sum(jnp.where(col == y_ref[...], lm, 0.0), axis=1,
                       keepdims=True)
        nll = lse - ysel
        rid = lax.broadcasted_iota(jnp.int32, (MB, 1), 0) + pl.program_id(0) * MB
        contrib = jnp.sum(jnp.where(rid < N, nll, 0.0), axis=(0, 1),
                          keepdims=True)

        @pl.when(pl.program_id(0) == 0)
        def _():
            ls_ref[...] = jnp.zeros((1, 1), jnp.float32)

        ls_ref[...] += contrib

    return pl.pallas_call(
        body,
        grid=(GRID_M,),
        in_specs=[
            pl.BlockSpec((2, MB, D_OP), lambda m: (0, m, 0)),
            pl.BlockSpec((MB, D_OP), lambda m: (m, 0)),
            pl.BlockSpec((MB, D_IN), lambda m: (m, 0)),
            pl.BlockSpec((1, D_OP), lambda m: (0, 0)),
            pl.BlockSpec((MB, 1), lambda m: (m, 0)),
        ],
        out_specs=[
            pl.BlockSpec((MB, D_OP), lambda m: (m, 0)),
            pl.BlockSpec((1, 1), lambda m: (0, 0)),
        ],
        out_shape=[
            jax.ShapeDtypeStruct((N, D_OP), jnp.float32),
            jax.ShapeDtypeStruct((1, 1), jnp.float32),
        ],
    )(p, hn4, dinv, b4p, y2)


def kernel(x, edge_index, y, W1, b1, W2, b2, W3, b3, W4, b4):
    src = edge_index[0]
    dst = edge_index[1]
    src80 = src.reshape(NSUB, 5, E // NSUB // 80 // 5, 80)
    dst80 = dst.reshape(NSUB, 5, E // NSUB // 80 // 5, 80)
    src40 = src.reshape(2 * NSUB, E // (2 * NSUB) // 40, 40)
    dst40 = dst.reshape(2 * NSUB, E // (2 * NSUB) // 40, 40)
    w4p = jnp.pad(W4, ((0, 0), (0, D_OP - D_OUT)))
    b4p = jnp.pad(b4, (0, D_OP - D_OUT)).reshape(1, D_OP)
    y2 = y.reshape(N, 1)

    degp = _sc_degree(dst40)
    dinv, xn = _tc_prep(degp, x)
    p0 = _sc_agg_split(xn, src40, dst40)
    hn1 = _tc_layer1(p0, xn, W1, b1, dinv)
    u1 = _sc_agg_chunks(hn1, src80, dst80, 128)
    hn2 = _tc_layer(u1, W2, b2, dinv)
    u2 = _sc_agg_chunks(hn2, src80, dst80, 128)
    hn4 = _tc_layer34(u2, W3, b3, w4p, dinv)
    p = _sc_agg_split(hn4, src40, dst40)
    lg, lsum = _tc_final(p, hn4, dinv, b4p, y2)
    logits = lg[:, :D_OUT]
    loss = lsum[0, 0] / jnp.float32(N)
    return logits, loss
